# baseline (speedup 1.0000x reference)
import functools

import jax
import jax.numpy as jnp
from jax import lax
from jax.experimental import pallas as pl
from jax.experimental.pallas import tpu as pltpu
from jax.experimental.pallas import tpu_sc as plsc

B = 16384
D = 64
K = 5
V = 100000
VP = V // 2
NFULL = V // 128
VTAIL = V - NFULL * 128
PTAIL = VTAIL // 2

NC = 2
NS = 16
NW = NC * NS
BPW = B // NW
GCH = 128
HB = 256
NCH = BPW // GCH
NGRP = HB // 16
VPAD = (NFULL + 1) * 64


def _transpose_kernel(ut_hbm, it_hbm, utail_hbm, itail_hbm,
                      upair_hbm, ipair_hbm, slab, obuf, sem_in, sem_out):
    wid = lax.axis_index("s") * NC + lax.axis_index("c")
    lane = lax.iota(jnp.int32, 16)
    perms = tuple((lane + s) & 15 for s in range(16))
    lane2 = 2 * lane
    nt = (NFULL - 1 - wid) // NW + 1

    @pl.when(wid == 0)
    def _():
        pltpu.sync_copy(utail_hbm, upair_hbm.at[pl.ds(NFULL * 64, PTAIL)])

    @pl.when(wid == 1)
    def _():
        pltpu.sync_copy(itail_hbm, ipair_hbm.at[pl.ds(NFULL * 64, PTAIL)])

    for src, dst in ((ut_hbm, upair_hbm), (it_hbm, ipair_hbm)):
        pltpu.async_copy(src.at[:, pl.ds(wid * 128, 128)], slab.at[0],
                         sem_in)

        def tile_body(ti, carry, src=src, dst=dst):
            t = wid + ti * NW
            buf = ti & 1

            @pl.when(ti + 1 < nt)
            def _():
                pltpu.async_copy(
                    src.at[:, pl.ds((t + NW) * 128, 128)],
                    slab.at[1 - buf], sem_in)

            pltpu.make_async_copy(src.at[:, pl.ds(0, 128)], slab.at[buf],
                                  sem_in).wait()

            @pl.when(ti >= 2)
            def _():
                pltpu.make_async_copy(obuf.at[buf], dst.at[pl.ds(0, 64)],
                                      sem_out).wait()

            bufv = jnp.full((16,), buf, jnp.int32)
            pr0 = lane

            @plsc.parallel_loop(0, 32, unroll=2)
            def blk_body(i):
                par = lax.shift_right_logical(i, 4)
                qc = lax.shift_right_logical(i, 2) & 3
                pb = i & 3
                d0 = qc * 16
                p0 = pb * 16
                rr0 = jnp.full((16,), 2 * p0 + par, jnp.int32) + lane2
                pr = p0 + pr0
                c0 = jnp.full((16,), par * D + d0, jnp.int32)
                for sh in range(16):
                    dv = d0 + perms[sh]
                    vals = plsc.load_gather(slab, [bufv, dv, rr0])
                    plsc.store_scatter(obuf, [bufv, pr, c0 + perms[sh]],
                                       vals)

            pltpu.async_copy(obuf.at[buf], dst.at[pl.ds(t * 64, 64)],
                             sem_out)
            return carry

        lax.fori_loop(0, nt, tile_body, 0)
        pltpu.make_async_copy(obuf.at[0], dst.at[pl.ds(0, 64)],
                              sem_out).wait()
        pltpu.make_async_copy(obuf.at[0], dst.at[pl.ds(0, 64)],
                              sem_out).wait()


def _gather_kernel(uidx_hbm, iidx_hbm, uemb_hbm, iemb_hbm, w_hbm, b_hbm,
                   out_hbm, idx_u, idx_i, pidx_u, pidx_i, u_rows, v_rows,
                   w_v, b_v, out_v, sem):
    wid = lax.axis_index("s") * NC + lax.axis_index("c")
    base = wid * BPW

    pltpu.sync_copy(uidx_hbm.at[wid], idx_u)
    pltpu.sync_copy(iidx_hbm.at[wid], idx_i)
    pltpu.sync_copy(w_hbm, w_v)
    pltpu.sync_copy(b_hbm, b_v)

    for j in range(NCH):
        for t in range(GCH // 16):
            sl = pl.ds(t * 16, 16)
            pidx_u[j, sl] = lax.shift_right_logical(idx_u[j, sl], 1)
            pidx_i[j, sl] = lax.shift_right_logical(idx_i[j, sl], 1)

    lane = lax.iota(jnp.int32, 16)
    m15 = lane == 15
    wv = [[w_v[pl.ds((k * 4 + c) * 16, 16)] for c in range(4)]
          for k in range(K)]
    bv = [b_v[pl.ds(k * 16, 16)] for k in range(K)]
    kvs = [jnp.full((16,), k, jnp.int32) for k in range(K)]

    for h in range(BPW // HB):
        copies = []
        for j in range(HB // GCH):
            jc = h * (HB // GCH) + j
            copies.append(pltpu.async_copy(
                uemb_hbm.at[pidx_u.at[jc]],
                u_rows.at[pl.ds(j * GCH, GCH)], sem))
            copies.append(pltpu.async_copy(
                iemb_hbm.at[pidx_i.at[jc]],
                v_rows.at[pl.ds(j * GCH, GCH)], sem))
        for c in copies:
            c.wait()

        @plsc.parallel_loop(0, HB, unroll=2)
        def row_body(r):
            jc16 = jnp.full((16,), h * (HB // GCH), jnp.int32) + \
                lax.shift_right_logical(r, 7)
            r16 = jnp.full((16,), r & 127, jnp.int32)
            up = plsc.load_gather(idx_u, [jc16, r16]) & 1
            ip = plsc.load_gather(idx_i, [jc16, r16]) & 1
            usel = up == 0
            isel = ip == 0
            t_ks = []
            for c in range(4):
                lo = u_rows[r, pl.ds(c * 16, 16)]
                hi = u_rows[r, pl.ds(D + c * 16, 16)]
                u_c = jnp.where(usel, lo, hi)
                lo = v_rows[r, pl.ds(c * 16, 16)]
                hi = v_rows[r, pl.ds(D + c * 16, 16)]
                v_c = jnp.where(isel, lo, hi)
                m_c = u_c * v_c
                for k in range(K):
                    t = m_c * wv[k][c]
                    t_ks.append(t) if c == 0 else None
                    if c:
                        t_ks[k] = t_ks[k] + t
            r16b = jnp.full((16,), r, jnp.int32)
            for k in range(K):
                sc = plsc.cumsum(t_ks[k]) + bv[k]
                p = 1.0 / (1.0 + jnp.exp(-sc))
                plsc.store_scatter(out_v, [r16b, kvs[k]], p, mask=m15)
        pltpu.sync_copy(out_v,
                        out_hbm.at[pl.ds(base + h * HB, HB)])


_SC_PARAMS = pltpu.CompilerParams(needs_layout_passes=False)


@jax.jit
def _bpr(uidx, iidx, ut, it, utail, itail, w_bc, b_bc):
    mesh = plsc.VectorSubcoreMesh(core_axis_name="c", subcore_axis_name="s")
    pair_sds = jax.ShapeDtypeStruct((VPAD, 2 * D), jnp.float32)
    k1 = functools.partial(
        pl.kernel,
        out_type=(pair_sds, pair_sds),
        mesh=mesh,
        compiler_params=_SC_PARAMS,
        scratch_types=[
            pltpu.VMEM((2, D, 128), jnp.float32),
            pltpu.VMEM((2, D, 128), jnp.float32),
            pltpu.SemaphoreType.DMA,
            pltpu.SemaphoreType.DMA,
        ],
    )(_transpose_kernel)
    upair, ipair = k1(ut, it, utail, itail)

    k2 = functools.partial(
        pl.kernel,
        out_type=jax.ShapeDtypeStruct((B, K), jnp.float32),
        mesh=mesh,
        compiler_params=_SC_PARAMS,
        scratch_types=[
            pltpu.VMEM((NCH, GCH), jnp.int32),
            pltpu.VMEM((NCH, GCH), jnp.int32),
            pltpu.VMEM((NCH, GCH), jnp.int32),
            pltpu.VMEM((NCH, GCH), jnp.int32),
            pltpu.VMEM((HB, 2 * D), jnp.float32),
            pltpu.VMEM((HB, 2 * D), jnp.float32),
            pltpu.VMEM((K * D,), jnp.float32),
            pltpu.VMEM((128,), jnp.float32),
            pltpu.VMEM((HB, K), jnp.float32),
            pltpu.SemaphoreType.DMA,
        ],
    )(_gather_kernel)
    return k2(uidx, iidx, upair, ipair, w_bc, b_bc)


def kernel(user_input, item_input, user_emb, item_emb, W, b):
    uidx = user_input.astype(jnp.int32).reshape(NW, NCH, GCH)
    iidx = item_input.astype(jnp.int32).reshape(NW, NCH, GCH)
    ut = user_emb.T
    it = item_emb.T
    utail = user_emb[NFULL * 128:].reshape(PTAIL, 2 * D)
    itail = item_emb[NFULL * 128:].reshape(PTAIL, 2 * D)
    w_bc = W.reshape(-1)
    b_bc = jnp.zeros((128,), jnp.float32).at[:K * 16].set(
        jnp.broadcast_to(b.reshape(K, 1), (K, 16)).reshape(-1))
    return _bpr(uidx, iidx, ut, it, utail, itail, w_bc, b_bc)


# speedup vs baseline: 1.2063x; 1.2063x over previous
import functools

import jax
import jax.numpy as jnp
from jax import lax
from jax.experimental import pallas as pl
from jax.experimental.pallas import tpu as pltpu
from jax.experimental.pallas import tpu_sc as plsc

B = 16384
D = 64
K = 5
V = 100000
VP = V // 2
NFULL = V // 128
VTAIL = V - NFULL * 128
PTAIL = VTAIL // 2

NC = 2
NS = 16
NW = NC * NS
BPW = B // NW
GCH = 128
HB = 256
NCH = BPW // GCH
NGRP = HB // 16
VPAD = (NFULL + 1) * 64


def _transpose_kernel(ut_hbm, it_hbm, utail_hbm, itail_hbm,
                      upair_hbm, ipair_hbm, slab, obuf, sem_in, sem_out):
    wid = lax.axis_index("s") * NC + lax.axis_index("c")
    lane = lax.iota(jnp.int32, 16)
    perms = tuple((lane + s) & 15 for s in range(16))
    lane2 = 2 * lane
    nt = (NFULL - 1 - wid) // NW + 1

    @pl.when(wid == 0)
    def _():
        pltpu.sync_copy(utail_hbm, upair_hbm.at[pl.ds(NFULL * 64, PTAIL)])

    @pl.when(wid == 1)
    def _():
        pltpu.sync_copy(itail_hbm, ipair_hbm.at[pl.ds(NFULL * 64, PTAIL)])

    for src, dst in ((ut_hbm, upair_hbm), (it_hbm, ipair_hbm)):
        pltpu.async_copy(src.at[:, pl.ds(wid * 128, 128)], slab.at[0],
                         sem_in)

        def tile_body(ti, carry, src=src, dst=dst):
            t = wid + ti * NW
            buf = ti & 1

            @pl.when(ti + 1 < nt)
            def _():
                pltpu.async_copy(
                    src.at[:, pl.ds((t + NW) * 128, 128)],
                    slab.at[1 - buf], sem_in)

            pltpu.make_async_copy(src.at[:, pl.ds(0, 128)], slab.at[buf],
                                  sem_in).wait()

            @pl.when(ti >= 2)
            def _():
                pltpu.make_async_copy(obuf.at[buf], dst.at[pl.ds(0, 64)],
                                      sem_out).wait()

            bufv = jnp.full((16,), buf, jnp.int32)
            pr0 = lane

            @plsc.parallel_loop(0, 32)
            def blk_body(i):
                par = lax.shift_right_logical(i, 4)
                qc = lax.shift_right_logical(i, 2) & 3
                pb = i & 3
                d0 = qc * 16
                p0 = pb * 16
                rr0 = jnp.full((16,), 2 * p0 + par, jnp.int32) + lane2
                pr = p0 + pr0
                c0 = jnp.full((16,), par * D + d0, jnp.int32)
                for sh in range(16):
                    dv = d0 + perms[sh]
                    vals = plsc.load_gather(slab, [bufv, dv, rr0])
                    plsc.store_scatter(obuf, [bufv, pr, c0 + perms[sh]],
                                       vals)

            pltpu.async_copy(obuf.at[buf], dst.at[pl.ds(t * 64, 64)],
                             sem_out)
            return carry

        lax.fori_loop(0, nt, tile_body, 0)
        pltpu.make_async_copy(obuf.at[0], dst.at[pl.ds(0, 64)],
                              sem_out).wait()
        pltpu.make_async_copy(obuf.at[0], dst.at[pl.ds(0, 64)],
                              sem_out).wait()


def _gather_kernel(uidx_hbm, iidx_hbm, uemb_hbm, iemb_hbm, w_hbm, b_hbm,
                   out_hbm, idx_u, idx_i, pidx_u, pidx_i, u_rows, v_rows,
                   w_v, b_v, out_v, sem):
    wid = lax.axis_index("s") * NC + lax.axis_index("c")
    base = wid * BPW

    pltpu.sync_copy(uidx_hbm.at[wid], idx_u)
    pltpu.sync_copy(iidx_hbm.at[wid], idx_i)
    pltpu.sync_copy(w_hbm, w_v)
    pltpu.sync_copy(b_hbm, b_v)

    for j in range(NCH):
        for t in range(GCH // 16):
            sl = pl.ds(t * 16, 16)
            pidx_u[j, sl] = lax.shift_right_logical(idx_u[j, sl], 1)
            pidx_i[j, sl] = lax.shift_right_logical(idx_i[j, sl], 1)

    lane = lax.iota(jnp.int32, 16)
    m15 = lane == 15
    wv = [[w_v[pl.ds((k * 4 + c) * 16, 16)] for c in range(4)]
          for k in range(K)]
    bv = [b_v[pl.ds(k * 16, 16)] for k in range(K)]
    kvs = [jnp.full((16,), k, jnp.int32) for k in range(K)]

    for h in range(BPW // HB):
        copies = []
        for j in range(HB // GCH):
            jc = h * (HB // GCH) + j
            copies.append(pltpu.async_copy(
                uemb_hbm.at[pidx_u.at[jc]],
                u_rows.at[pl.ds(j * GCH, GCH)], sem))
            copies.append(pltpu.async_copy(
                iemb_hbm.at[pidx_i.at[jc]],
                v_rows.at[pl.ds(j * GCH, GCH)], sem))
        for c in copies:
            c.wait()

        @plsc.parallel_loop(0, HB)
        def row_body(r):
            jc16 = jnp.full((16,), h * (HB // GCH), jnp.int32) + \
                lax.shift_right_logical(r, 7)
            r16 = jnp.full((16,), r & 127, jnp.int32)
            up = plsc.load_gather(idx_u, [jc16, r16]) & 1
            ip = plsc.load_gather(idx_i, [jc16, r16]) & 1
            usel = up == 0
            isel = ip == 0
            t_ks = []
            for c in range(4):
                lo = u_rows[r, pl.ds(c * 16, 16)]
                hi = u_rows[r, pl.ds(D + c * 16, 16)]
                u_c = jnp.where(usel, lo, hi)
                lo = v_rows[r, pl.ds(c * 16, 16)]
                hi = v_rows[r, pl.ds(D + c * 16, 16)]
                v_c = jnp.where(isel, lo, hi)
                m_c = u_c * v_c
                for k in range(K):
                    t = m_c * wv[k][c]
                    t_ks.append(t) if c == 0 else None
                    if c:
                        t_ks[k] = t_ks[k] + t
            r16b = jnp.full((16,), r, jnp.int32)
            for k in range(K):
                sc = plsc.cumsum(t_ks[k]) + bv[k]
                p = 1.0 / (1.0 + jnp.exp(-sc))
                plsc.store_scatter(out_v, [r16b, kvs[k]], p, mask=m15)
        pltpu.sync_copy(out_v,
                        out_hbm.at[pl.ds(base + h * HB, HB)])


_SC_PARAMS = pltpu.CompilerParams(needs_layout_passes=False)


@jax.jit
def _bpr(uidx, iidx, ut, it, utail, itail, w_bc, b_bc):
    mesh = plsc.VectorSubcoreMesh(core_axis_name="c", subcore_axis_name="s")
    pair_sds = jax.ShapeDtypeStruct((VPAD, 2 * D), jnp.float32)
    k1 = functools.partial(
        pl.kernel,
        out_type=(pair_sds, pair_sds),
        mesh=mesh,
        compiler_params=_SC_PARAMS,
        scratch_types=[
            pltpu.VMEM((2, D, 128), jnp.float32),
            pltpu.VMEM((2, D, 128), jnp.float32),
            pltpu.SemaphoreType.DMA,
            pltpu.SemaphoreType.DMA,
        ],
    )(_transpose_kernel)
    upair, ipair = k1(ut, it, utail, itail)

    k2 = functools.partial(
        pl.kernel,
        out_type=jax.ShapeDtypeStruct((B, K), jnp.float32),
        mesh=mesh,
        compiler_params=_SC_PARAMS,
        scratch_types=[
            pltpu.VMEM((NCH, GCH), jnp.int32),
            pltpu.VMEM((NCH, GCH), jnp.int32),
            pltpu.VMEM((NCH, GCH), jnp.int32),
            pltpu.VMEM((NCH, GCH), jnp.int32),
            pltpu.VMEM((HB, 2 * D), jnp.float32),
            pltpu.VMEM((HB, 2 * D), jnp.float32),
            pltpu.VMEM((K * D,), jnp.float32),
            pltpu.VMEM((128,), jnp.float32),
            pltpu.VMEM((HB, K), jnp.float32),
            pltpu.SemaphoreType.DMA,
        ],
    )(_gather_kernel)
    return k2(uidx, iidx, upair, ipair, w_bc, b_bc)


def kernel(user_input, item_input, user_emb, item_emb, W, b):
    uidx = user_input.astype(jnp.int32).reshape(NW, NCH, GCH)
    iidx = item_input.astype(jnp.int32).reshape(NW, NCH, GCH)
    ut = user_emb.T
    it = item_emb.T
    utail = user_emb[NFULL * 128:].reshape(PTAIL, 2 * D)
    itail = item_emb[NFULL * 128:].reshape(PTAIL, 2 * D)
    w_bc = W.reshape(-1)
    b_bc = jnp.zeros((128,), jnp.float32).at[:K * 16].set(
        jnp.broadcast_to(b.reshape(K, 1), (K, 16)).reshape(-1))
    return _bpr(uidx, iidx, ut, it, utail, itail, w_bc, b_bc)
